# baseline (speedup 1.0000x reference)
import functools

import jax
import jax.numpy as jnp
from jax import lax
from jax.experimental import pallas as pl
from jax.experimental.pallas import tpu as pltpu
from jax.experimental.pallas import tpu_sc as plsc

N, E, D, G, DIM_O = 10000, 320000, 128, 64, 10
NP = 10240
NW = 32
CHUNK = 128
CHUNKS_PER_W = 80
NBUF = 4
EPW = CHUNK * CHUNKS_PER_W
EP = NW * EPW
ROWS_PER_SUB = NP // 16

_INT_MAX = 2147483647
_ONE_KEY = 1065353216


def _sc_agg_body(h_hbm, src_hbm, dst_hbm, zero_hbm, out_hbm,
                 isrc, idst, rows, aggsh, sem):
    c = lax.axis_index("c")
    s = lax.axis_index("s")
    w = c * 16 + s
    pltpu.sync_copy(zero_hbm.at[pl.ds(s * ROWS_PER_SUB, ROWS_PER_SUB)],
                    aggsh.at[pl.ds(s * ROWS_PER_SUB, ROWS_PER_SUB)])
    plsc.subcore_barrier()

    def step(j, carry):
        pltpu.sync_copy(src_hbm.at[w, j], isrc)
        pltpu.sync_copy(dst_hbm.at[w, j], idst)
        pltpu.async_copy(h_hbm.at[isrc], rows, sem).wait()
        pltpu.sync_copy(rows, aggsh.at[idst], add=True)
        return carry

    lax.fori_loop(0, CHUNKS_PER_W, step, 0)
    plsc.subcore_barrier()
    pltpu.sync_copy(aggsh.at[pl.ds(s * ROWS_PER_SUB, ROWS_PER_SUB)],
                    out_hbm.at[c, pl.ds(s * ROWS_PER_SUB, ROWS_PER_SUB)])


_SC_AGG_CACHE = []


def _sc_agg(h, srcp, dstp, zeros):
    if not _SC_AGG_CACHE:
        _SC_AGG_CACHE.append(functools.partial(
            pl.kernel,
            mesh=plsc.VectorSubcoreMesh(core_axis_name="c",
                                        subcore_axis_name="s"),
            out_type=jax.ShapeDtypeStruct((2, NP, D), jnp.float32),
            scratch_types=[
                pltpu.VMEM((CHUNK,), jnp.int32),
                pltpu.VMEM((CHUNK,), jnp.int32),
                pltpu.VMEM((CHUNK, D), jnp.float32),
                pltpu.VMEM_SHARED((NP, D), jnp.float32),
                pltpu.SemaphoreType.DMA,
            ],
        )(_sc_agg_body))
    return _SC_AGG_CACHE[0](h, srcp, dstp, zeros)


def _conv_body(a0_ref, a1_ref, h_ref, wr_ref, wo_ref, br_ref,
               m_ref, v_ref, w_ref, b_ref, wn_ref, norm_ref,
               y_ref, s_ref):
    agg = (a0_ref[...] + a1_ref[...]).astype(jnp.bfloat16)
    out = (jnp.dot(agg, wr_ref[...], preferred_element_type=jnp.float32)
           + br_ref[...]
           + jnp.dot(h_ref[...].astype(jnp.bfloat16), wo_ref[...],
                     preferred_element_type=jnp.float32))
    hc = jnp.maximum(out, 0.0)
    y = ((hc - m_ref[...]) / jnp.sqrt(v_ref[...] + 1e-5)
         * w_ref[...] + b_ref[...])
    y_ref[...] = y
    sv = jnp.dot(y.astype(jnp.bfloat16), wn_ref[...],
                 preferred_element_type=jnp.float32)
    s = jnp.tanh(sv / norm_ref[0, 0])
    s_ref[...] = s + 0.0


def _conv(a0, a1, h, wrT, woT, br, bnm, bnv, bnw, bnb, wn, norm):
    grid = NP // 256
    row = pl.BlockSpec((256, D), lambda i: (i, 0))
    col = pl.BlockSpec((256, 1), lambda i: (i, 0))
    full = pl.BlockSpec((D, D), lambda i: (0, 0))
    vec = pl.BlockSpec((1, D), lambda i: (0, 0))
    wnv = pl.BlockSpec((D, 1), lambda i: (0, 0))
    one = pl.BlockSpec((1, 1), lambda i: (0, 0))
    return pl.pallas_call(
        _conv_body,
        grid=(grid,),
        in_specs=[row, row, row, full, full, vec, vec, vec, vec, vec,
                  wnv, one],
        out_specs=(row, col),
        out_shape=(jax.ShapeDtypeStruct((NP, D), jnp.float32),
                   jax.ShapeDtypeStruct((NP, 1), jnp.float32)),
    )(a0, a1, h, wrT, woT, br, bnm, bnv, bnw, bnb, wn, norm)


def _select_body(s_ref, nm_ref, b_ref, sel_ref, nmo_ref):
    s = s_ref[...]
    nm = nm_ref[...]
    bcol = b_ref[...]

    bits = lax.bitcast_convert_type(s, jnp.int32)
    asc = jnp.where(bits < 0, bits ^ jnp.int32(0x7FFFFFFF), bits)
    alive = nm > 0.0
    d = jnp.where(alive, jnp.int32(_ONE_KEY) - asc,
                  jnp.int32(_INT_MAX))

    gi = lax.broadcasted_iota(jnp.int32, (1, G), 1)
    oh = bcol == gi
    oh_alive = jnp.where(oh & alive, 1, 0)
    cnt = jnp.sum(oh_alive, axis=0, keepdims=True)
    k = (cnt + 1) // 2

    def bit_step(j, t):
        cand = t + lax.shift_left(jnp.int32(1), 30 - j)
        pred = jnp.where((d < cand) & oh, 1, 0)
        c1 = jnp.sum(pred, axis=0, keepdims=True)
        return jnp.where(c1 >= k, t, cand)

    T = lax.fori_loop(0, 31, bit_step, jnp.zeros((1, G), jnp.int32))

    strict_m = (d < T) & oh
    n_lt = jnp.sum(jnp.where(strict_m, 1, 0), axis=0, keepdims=True)
    eq_m = (d == T) & oh
    need = k - n_lt
    idx = lax.broadcasted_iota(jnp.int32, (NP, 1), 0)

    def tie_step(j, t):
        cand = t + lax.shift_left(jnp.int32(1), 13 - j)
        pred = jnp.where(eq_m & (idx < cand), 1, 0)
        c2 = jnp.sum(pred, axis=0, keepdims=True)
        return jnp.where(c2 >= need, t, cand)

    T2 = lax.fori_loop(0, 14, tie_step, jnp.zeros((1, G), jnp.int32))
    tie_m = eq_m & (idx <= T2) & (need > 0)
    keep_m = strict_m | tie_m
    keep = jnp.max(jnp.where(keep_m, 1, 0), axis=1, keepdims=True) > 0

    sel_ref[...] = jnp.where(keep, s, 0.0)
    nmo_ref[...] = jnp.where(keep, 1.0, 0.0)


def _select(s, nm, bcol):
    return pl.pallas_call(
        _select_body,
        out_shape=(
            jax.ShapeDtypeStruct((NP, 1), jnp.float32),
            jax.ShapeDtypeStruct((NP, 1), jnp.float32),
        ),
        compiler_params=pltpu.CompilerParams(
            vmem_limit_bytes=100 * 1024 * 1024),
    )(s, nm, bcol)


def _readout_body(y_ref, sel_ref, nmo_ref, b_ref, ohT_ref,
                  h2_ref, read_ref):
    y = y_ref[...]
    sel = sel_ref[...]
    keep_f = nmo_ref[...]
    bcol = b_ref[...]
    ohT = ohT_ref[...]
    h2 = y * sel
    h2_ref[...] = h2
    keep = keep_f > 0.0

    sm = jnp.dot(ohT, h2, preferred_element_type=jnp.float32,
                 precision=jax.lax.Precision.HIGHEST)
    kcol = jnp.dot(ohT, keep_f, preferred_element_type=jnp.float32,
                   precision=jax.lax.Precision.HIGHEST)
    mean = sm / jnp.maximum(kcol, 1.0)

    g64 = lax.broadcasted_iota(jnp.int32, (G, 1), 0)

    def max_step(g, mx):
        m = keep & (bcol == g)
        hm = jnp.where(m, h2, -1e30)
        row = jnp.max(hm, axis=0, keepdims=True)
        return jnp.where(g64 == g, row, mx)

    mx = lax.fori_loop(0, G, max_step, jnp.full((G, D), -1e30, jnp.float32))
    mx = jnp.where(kcol > 0.0, mx, 0.0)
    read_ref[:, 0:D] = mx
    read_ref[:, D:2 * D] = mean


def _readout(y, sel, nmo, bcol, ohT):
    return pl.pallas_call(
        _readout_body,
        out_shape=(
            jax.ShapeDtypeStruct((NP, D), jnp.float32),
            jax.ShapeDtypeStruct((G, 2 * D), jnp.float32),
        ),
        compiler_params=pltpu.CompilerParams(
            vmem_limit_bytes=100 * 1024 * 1024),
    )(y, sel, nmo, bcol, ohT)


def _head_body(r1, r2, r3, r4, r5, w1, b1, w2, b2, w3, b3, o_ref):
    bf = jnp.bfloat16
    z = r1[...] + r2[...] + r3[...] + r4[...] + r5[...]
    z = jnp.maximum(jnp.dot(z.astype(bf), w1[...],
                            preferred_element_type=jnp.float32)
                    + b1[...], 0.0)
    z = jnp.maximum(jnp.dot(z.astype(bf), w2[...],
                            preferred_element_type=jnp.float32)
                    + b2[...], 0.0)
    z = jnp.dot(z.astype(bf), w3[...],
                preferred_element_type=jnp.float32) + b3[...]
    col = lax.broadcasted_iota(jnp.int32, (G, D), 1)
    zm = jnp.where(col < DIM_O, z, -1e30)
    m = jnp.max(zm, axis=1, keepdims=True)
    ex = jnp.where(col < DIM_O, jnp.exp(zm - m), 0.0)
    lse = jnp.log(jnp.sum(ex, axis=1, keepdims=True))
    o_ref[...] = zm - m - lse


def _head(reads, w1T, b1, w2T, b2, w3T, b3):
    return pl.pallas_call(
        _head_body,
        out_shape=jax.ShapeDtypeStruct((G, D), jnp.float32),
    )(*reads, w1T, b1, w2T, b2, w3T, b3)


def kernel(x, params, edge_index, batch):
    f32 = jnp.float32
    xp = jnp.pad(x, ((0, NP - N), (0, 0)))
    bcol = jnp.pad(batch, (0, NP - N)).astype(jnp.int32).reshape(NP, 1)
    nm = jnp.pad(jnp.ones((N, 1), f32), ((0, NP - N), (0, 0)))
    ohT = (bcol.reshape(1, NP) == jnp.arange(G, dtype=jnp.int32)
           .reshape(G, 1)).astype(f32)
    srcp = jnp.pad(edge_index[0], (0, EP - E),
                   constant_values=N).reshape(NW, CHUNKS_PER_W, CHUNK)
    dstp = jnp.pad(edge_index[1], (0, EP - E),
                   constant_values=N).reshape(NW, CHUNKS_PER_W, CHUNK)
    zeros = jnp.zeros((NP, D), f32)

    bf = jnp.bfloat16
    h = xp
    reads = []
    for i in range(5):
        p = params['conv'][i]
        bn = params['bn'][i]
        w = params['pool_w'][i]
        wn = w.astype(bf).reshape(D, 1)
        norm = jnp.linalg.norm(w).reshape(1, 1)

        parts = _sc_agg(h, srcp, dstp, zeros)
        y, s = _conv(parts[0], parts[1], h,
                     p['Wr'].T.astype(bf), p['Wo'].T.astype(bf),
                     p['br'].reshape(1, D), bn['m'].reshape(1, D),
                     bn['v'].reshape(1, D), bn['w'].reshape(1, D),
                     bn['b'].reshape(1, D), wn, norm)
        sel, nmo = _select(s, nm, bcol)
        h, rd = _readout(y, sel, nmo, bcol, ohT)
        nm = nmo
        reads.append(rd)

    l1, l2, l3 = params['lin1'], params['lin2'], params['lin3']
    w2T = jnp.zeros((D, D), f32).at[:, :64].set(l2['W'].T)
    b2 = jnp.zeros((1, D), f32).at[0, :64].set(l2['b'])
    w3T = jnp.zeros((D, D), f32).at[:64, :DIM_O].set(l3['W'].T)
    b3 = jnp.zeros((1, D), f32).at[0, :DIM_O].set(l3['b'])
    out = _head(reads, l1['W'].T.astype(bf), l1['b'].reshape(1, D),
                w2T.astype(bf), b2, w3T.astype(bf), b3)
    return out[:, :DIM_O]


# speedup vs baseline: 1.0002x; 1.0002x over previous
import functools

import jax
import jax.numpy as jnp
from jax import lax
from jax.experimental import pallas as pl
from jax.experimental.pallas import tpu as pltpu
from jax.experimental.pallas import tpu_sc as plsc

N, E, D, G, DIM_O = 10000, 320000, 128, 64, 10
NP = 10240
NW = 32
CHUNK = 128
CHUNKS_PER_W = 80
NBUF = 4
EPW = CHUNK * CHUNKS_PER_W
EP = NW * EPW
ROWS_PER_SUB = NP // 16

_INT_MAX = 2147483647
_ONE_KEY = 1065353216


def _sc_agg_body(h_hbm, src_hbm, dst_hbm, zero_hbm, out_hbm,
                 isrc, idst, rows, aggsh, sem):
    c = lax.axis_index("c")
    s = lax.axis_index("s")
    w = c * 16 + s
    pltpu.sync_copy(zero_hbm.at[pl.ds(s * ROWS_PER_SUB, ROWS_PER_SUB)],
                    aggsh.at[pl.ds(s * ROWS_PER_SUB, ROWS_PER_SUB)])
    plsc.subcore_barrier()
    base0 = w * EPW

    def step(j, carry):
        b = base0 + j * CHUNK
        pltpu.sync_copy(src_hbm.at[pl.ds(b, CHUNK)], isrc)
        pltpu.sync_copy(dst_hbm.at[pl.ds(b, CHUNK)], idst)
        pltpu.async_copy(h_hbm.at[isrc], rows, sem).wait()
        pltpu.sync_copy(rows, aggsh.at[idst], add=True)
        return carry

    lax.fori_loop(0, CHUNKS_PER_W, step, 0)
    plsc.subcore_barrier()
    pltpu.sync_copy(aggsh.at[pl.ds(s * ROWS_PER_SUB, ROWS_PER_SUB)],
                    out_hbm.at[c, pl.ds(s * ROWS_PER_SUB, ROWS_PER_SUB)])


_SC_AGG_CACHE = []


def _sc_agg(h, srcp, dstp, zeros):
    if not _SC_AGG_CACHE:
        _SC_AGG_CACHE.append(functools.partial(
            pl.kernel,
            mesh=plsc.VectorSubcoreMesh(core_axis_name="c",
                                        subcore_axis_name="s"),
            out_type=jax.ShapeDtypeStruct((2, NP, D), jnp.float32),
            scratch_types=[
                pltpu.VMEM((CHUNK,), jnp.int32),
                pltpu.VMEM((CHUNK,), jnp.int32),
                pltpu.VMEM((CHUNK, D), jnp.float32),
                pltpu.VMEM_SHARED((NP, D), jnp.float32),
                pltpu.SemaphoreType.DMA,
            ],
        )(_sc_agg_body))
    return _SC_AGG_CACHE[0](h, srcp, dstp, zeros)


def _conv_body(a0_ref, a1_ref, h_ref, wr_ref, wo_ref, br_ref,
               m_ref, v_ref, w_ref, b_ref, wn_ref, norm_ref,
               y_ref, s_ref):
    agg = (a0_ref[...] + a1_ref[...]).astype(jnp.bfloat16)
    out = (jnp.dot(agg, wr_ref[...], preferred_element_type=jnp.float32)
           + br_ref[...]
           + jnp.dot(h_ref[...].astype(jnp.bfloat16), wo_ref[...],
                     preferred_element_type=jnp.float32))
    hc = jnp.maximum(out, 0.0)
    y = ((hc - m_ref[...]) / jnp.sqrt(v_ref[...] + 1e-5)
         * w_ref[...] + b_ref[...])
    y_ref[...] = y
    sv = jnp.dot(y.astype(jnp.bfloat16), wn_ref[...],
                 preferred_element_type=jnp.float32)
    s = jnp.tanh(sv / norm_ref[0, 0])
    s_ref[...] = s + 0.0


def _conv(a0, a1, h, wrT, woT, br, bnm, bnv, bnw, bnb, wn, norm):
    grid = NP // 256
    row = pl.BlockSpec((256, D), lambda i: (i, 0))
    col = pl.BlockSpec((256, 1), lambda i: (i, 0))
    full = pl.BlockSpec((D, D), lambda i: (0, 0))
    vec = pl.BlockSpec((1, D), lambda i: (0, 0))
    wnv = pl.BlockSpec((D, 1), lambda i: (0, 0))
    one = pl.BlockSpec((1, 1), lambda i: (0, 0))
    return pl.pallas_call(
        _conv_body,
        grid=(grid,),
        in_specs=[row, row, row, full, full, vec, vec, vec, vec, vec,
                  wnv, one],
        out_specs=(row, col),
        out_shape=(jax.ShapeDtypeStruct((NP, D), jnp.float32),
                   jax.ShapeDtypeStruct((NP, 1), jnp.float32)),
    )(a0, a1, h, wrT, woT, br, bnm, bnv, bnw, bnb, wn, norm)


def _select_body(s_ref, nm_ref, b_ref, sel_ref, nmo_ref):
    s = s_ref[...]
    nm = nm_ref[...]
    bcol = b_ref[...]

    bits = lax.bitcast_convert_type(s, jnp.int32)
    asc = jnp.where(bits < 0, bits ^ jnp.int32(0x7FFFFFFF), bits)
    alive = nm > 0.0
    d = jnp.where(alive, jnp.int32(_ONE_KEY) - asc,
                  jnp.int32(_INT_MAX))

    gi = lax.broadcasted_iota(jnp.int32, (1, G), 1)
    oh = bcol == gi
    oh_alive = jnp.where(oh & alive, 1, 0)
    cnt = jnp.sum(oh_alive, axis=0, keepdims=True)
    k = (cnt + 1) // 2

    def bit_step(j, t):
        cand = t + lax.shift_left(jnp.int32(1), 30 - j)
        pred = jnp.where((d < cand) & oh, 1, 0)
        c1 = jnp.sum(pred, axis=0, keepdims=True)
        return jnp.where(c1 >= k, t, cand)

    T = lax.fori_loop(0, 31, bit_step, jnp.zeros((1, G), jnp.int32))

    strict_m = (d < T) & oh
    n_lt = jnp.sum(jnp.where(strict_m, 1, 0), axis=0, keepdims=True)
    eq_m = (d == T) & oh
    need = k - n_lt
    idx = lax.broadcasted_iota(jnp.int32, (NP, 1), 0)

    def tie_step(j, t):
        cand = t + lax.shift_left(jnp.int32(1), 13 - j)
        pred = jnp.where(eq_m & (idx < cand), 1, 0)
        c2 = jnp.sum(pred, axis=0, keepdims=True)
        return jnp.where(c2 >= need, t, cand)

    T2 = lax.fori_loop(0, 14, tie_step, jnp.zeros((1, G), jnp.int32))
    tie_m = eq_m & (idx <= T2) & (need > 0)
    keep_m = strict_m | tie_m
    keep = jnp.max(jnp.where(keep_m, 1, 0), axis=1, keepdims=True) > 0

    sel_ref[...] = jnp.where(keep, s, 0.0)
    nmo_ref[...] = jnp.where(keep, 1.0, 0.0)


def _select(s, nm, bcol):
    return pl.pallas_call(
        _select_body,
        out_shape=(
            jax.ShapeDtypeStruct((NP, 1), jnp.float32),
            jax.ShapeDtypeStruct((NP, 1), jnp.float32),
        ),
        compiler_params=pltpu.CompilerParams(
            vmem_limit_bytes=100 * 1024 * 1024),
    )(s, nm, bcol)


def _readout_body(y_ref, sel_ref, nmo_ref, b_ref, ohT_ref,
                  h2_ref, read_ref):
    y = y_ref[...]
    sel = sel_ref[...]
    keep_f = nmo_ref[...]
    bcol = b_ref[...]
    ohT = ohT_ref[...]
    h2 = y * sel
    h2_ref[...] = h2
    keep = keep_f > 0.0

    sm = jnp.dot(ohT, h2, preferred_element_type=jnp.float32,
                 precision=jax.lax.Precision.HIGHEST)
    kcol = jnp.dot(ohT, keep_f, preferred_element_type=jnp.float32,
                   precision=jax.lax.Precision.HIGHEST)
    mean = sm / jnp.maximum(kcol, 1.0)

    g64 = lax.broadcasted_iota(jnp.int32, (G, 1), 0)

    def max_step(g, mx):
        m = keep & (bcol == g)
        hm = jnp.where(m, h2, -1e30)
        row = jnp.max(hm, axis=0, keepdims=True)
        return jnp.where(g64 == g, row, mx)

    mx = lax.fori_loop(0, G, max_step, jnp.full((G, D), -1e30, jnp.float32))
    mx = jnp.where(kcol > 0.0, mx, 0.0)
    read_ref[:, 0:D] = mx
    read_ref[:, D:2 * D] = mean


def _readout(y, sel, nmo, bcol, ohT):
    return pl.pallas_call(
        _readout_body,
        out_shape=(
            jax.ShapeDtypeStruct((NP, D), jnp.float32),
            jax.ShapeDtypeStruct((G, 2 * D), jnp.float32),
        ),
        compiler_params=pltpu.CompilerParams(
            vmem_limit_bytes=100 * 1024 * 1024),
    )(y, sel, nmo, bcol, ohT)


def _head_body(r1, r2, r3, r4, r5, w1, b1, w2, b2, w3, b3, o_ref):
    bf = jnp.bfloat16
    z = r1[...] + r2[...] + r3[...] + r4[...] + r5[...]
    z = jnp.maximum(jnp.dot(z.astype(bf), w1[...],
                            preferred_element_type=jnp.float32)
                    + b1[...], 0.0)
    z = jnp.maximum(jnp.dot(z.astype(bf), w2[...],
                            preferred_element_type=jnp.float32)
                    + b2[...], 0.0)
    z = jnp.dot(z.astype(bf), w3[...],
                preferred_element_type=jnp.float32) + b3[...]
    col = lax.broadcasted_iota(jnp.int32, (G, D), 1)
    zm = jnp.where(col < DIM_O, z, -1e30)
    m = jnp.max(zm, axis=1, keepdims=True)
    ex = jnp.where(col < DIM_O, jnp.exp(zm - m), 0.0)
    lse = jnp.log(jnp.sum(ex, axis=1, keepdims=True))
    o_ref[...] = zm - m - lse


def _head(reads, w1T, b1, w2T, b2, w3T, b3):
    return pl.pallas_call(
        _head_body,
        out_shape=jax.ShapeDtypeStruct((G, D), jnp.float32),
    )(*reads, w1T, b1, w2T, b2, w3T, b3)


def kernel(x, params, edge_index, batch):
    f32 = jnp.float32
    xp = jnp.pad(x, ((0, NP - N), (0, 0)))
    bcol = jnp.pad(batch, (0, NP - N)).astype(jnp.int32).reshape(NP, 1)
    nm = jnp.pad(jnp.ones((N, 1), f32), ((0, NP - N), (0, 0)))
    ohT = (bcol.reshape(1, NP) == jnp.arange(G, dtype=jnp.int32)
           .reshape(G, 1)).astype(f32)
    srcp = jnp.pad(edge_index[0], (0, EP - E), constant_values=N)
    dstp = jnp.pad(edge_index[1], (0, EP - E), constant_values=N)
    zeros = jnp.zeros((NP, D), f32)

    bf = jnp.bfloat16
    h = xp
    reads = []
    for i in range(5):
        p = params['conv'][i]
        bn = params['bn'][i]
        w = params['pool_w'][i]
        wn = w.astype(bf).reshape(D, 1)
        norm = jnp.linalg.norm(w).reshape(1, 1)

        parts = _sc_agg(h, srcp, dstp, zeros)
        y, s = _conv(parts[0], parts[1], h,
                     p['Wr'].T.astype(bf), p['Wo'].T.astype(bf),
                     p['br'].reshape(1, D), bn['m'].reshape(1, D),
                     bn['v'].reshape(1, D), bn['w'].reshape(1, D),
                     bn['b'].reshape(1, D), wn, norm)
        sel, nmo = _select(s, nm, bcol)
        h, rd = _readout(y, sel, nmo, bcol, ohT)
        nm = nmo
        reads.append(rd)

    l1, l2, l3 = params['lin1'], params['lin2'], params['lin3']
    w2T = jnp.zeros((D, D), f32).at[:, :64].set(l2['W'].T)
    b2 = jnp.zeros((1, D), f32).at[0, :64].set(l2['b'])
    w3T = jnp.zeros((D, D), f32).at[:64, :DIM_O].set(l3['W'].T)
    b3 = jnp.zeros((1, D), f32).at[0, :DIM_O].set(l3['b'])
    out = _head(reads, l1['W'].T.astype(bf), l1['b'].reshape(1, D),
                w2T.astype(bf), b2, w3T.astype(bf), b3)
    return out[:, :DIM_O]


# speedup vs baseline: 1.3250x; 1.3247x over previous
import functools

import jax
import jax.numpy as jnp
from jax import lax
from jax.experimental import pallas as pl
from jax.experimental.pallas import tpu as pltpu
from jax.experimental.pallas import tpu_sc as plsc

N, E, D, G, DIM_O = 10000, 320000, 128, 64, 10
NP = 10240
NW = 32
CHUNK = 128
CHUNKS_PER_W = 79
EPW = CHUNK * CHUNKS_PER_W
EP = NW * EPW
ROWS_PER_SUB = NP // 16

_INT_MAX = 2147483647
_ONE_KEY = 1065353216


def _sc_agg_body(h_hbm, src_hbm, dst_hbm, zero_hbm, out_hbm,
                 isrc, idst, rows, aggsh, sem):
    c = lax.axis_index("c")
    s = lax.axis_index("s")
    w = c * 16 + s
    pltpu.sync_copy(zero_hbm.at[pl.ds(s * ROWS_PER_SUB, ROWS_PER_SUB)],
                    aggsh.at[pl.ds(s * ROWS_PER_SUB, ROWS_PER_SUB)])
    plsc.subcore_barrier()
    base0 = w * EPW

    def step(j, carry):
        b = base0 + j * CHUNK
        pltpu.sync_copy(src_hbm.at[pl.ds(b, CHUNK)], isrc)
        pltpu.sync_copy(dst_hbm.at[pl.ds(b, CHUNK)], idst)
        pltpu.async_copy(h_hbm.at[isrc], rows, sem).wait()
        pltpu.sync_copy(rows, aggsh.at[idst], add=True)
        return carry

    lax.fori_loop(0, CHUNKS_PER_W, step, 0)
    plsc.subcore_barrier()
    pltpu.sync_copy(aggsh.at[pl.ds(s * ROWS_PER_SUB, ROWS_PER_SUB)],
                    out_hbm.at[c, pl.ds(s * ROWS_PER_SUB, ROWS_PER_SUB)])


_SC_AGG_CACHE = []


def _sc_agg(h, srcp, dstp, zeros):
    if not _SC_AGG_CACHE:
        _SC_AGG_CACHE.append(functools.partial(
            pl.kernel,
            mesh=plsc.VectorSubcoreMesh(core_axis_name="c",
                                        subcore_axis_name="s"),
            out_type=jax.ShapeDtypeStruct((2, NP, D), jnp.float32),
            scratch_types=[
                pltpu.VMEM((CHUNK,), jnp.int32),
                pltpu.VMEM((CHUNK,), jnp.int32),
                pltpu.VMEM((CHUNK, D), jnp.float32),
                pltpu.VMEM_SHARED((NP, D), jnp.float32),
                pltpu.SemaphoreType.DMA,
            ],
        )(_sc_agg_body))
    return _SC_AGG_CACHE[0](h, srcp, dstp, zeros)


def _conv_body(a0_ref, a1_ref, h_ref, wr_ref, wo_ref, br_ref,
               m_ref, v_ref, w_ref, b_ref, wn_ref, norm_ref,
               y_ref, s_ref):
    agg = (a0_ref[...] + a1_ref[...]).astype(jnp.bfloat16)
    out = (jnp.dot(agg, wr_ref[...], preferred_element_type=jnp.float32)
           + br_ref[...]
           + jnp.dot(h_ref[...].astype(jnp.bfloat16), wo_ref[...],
                     preferred_element_type=jnp.float32))
    hc = jnp.maximum(out, 0.0)
    y = ((hc - m_ref[...]) / jnp.sqrt(v_ref[...] + 1e-5)
         * w_ref[...] + b_ref[...])
    y_ref[...] = y
    sv = jnp.dot(y.astype(jnp.bfloat16), wn_ref[...],
                 preferred_element_type=jnp.float32)
    s = jnp.tanh(sv / norm_ref[0, 0])
    s_ref[...] = s + 0.0


def _conv(a0, a1, h, wrT, woT, br, bnm, bnv, bnw, bnb, wn, norm):
    grid = NP // 256
    row = pl.BlockSpec((256, D), lambda i: (i, 0))
    col = pl.BlockSpec((256, 1), lambda i: (i, 0))
    full = pl.BlockSpec((D, D), lambda i: (0, 0))
    vec = pl.BlockSpec((1, D), lambda i: (0, 0))
    wnv = pl.BlockSpec((D, 1), lambda i: (0, 0))
    one = pl.BlockSpec((1, 1), lambda i: (0, 0))
    return pl.pallas_call(
        _conv_body,
        grid=(grid,),
        in_specs=[row, row, row, full, full, vec, vec, vec, vec, vec,
                  wnv, one],
        out_specs=(row, col),
        out_shape=(jax.ShapeDtypeStruct((NP, D), jnp.float32),
                   jax.ShapeDtypeStruct((NP, 1), jnp.float32)),
    )(a0, a1, h, wrT, woT, br, bnm, bnv, bnw, bnb, wn, norm)


def _select_body(s_ref, nm_ref, b_ref, sel_ref, nmo_ref):
    s = s_ref[...]
    nm = nm_ref[...]
    bcol = b_ref[...]

    bits = lax.bitcast_convert_type(s, jnp.int32)
    asc = jnp.where(bits < 0, bits ^ jnp.int32(0x7FFFFFFF), bits)
    alive = nm > 0.0
    d = jnp.where(alive, jnp.int32(_ONE_KEY) - asc,
                  jnp.int32(_INT_MAX))

    gi = lax.broadcasted_iota(jnp.int32, (1, G), 1)
    oh = bcol == gi
    oh_alive = jnp.where(oh & alive, 1, 0)
    cnt = jnp.sum(oh_alive, axis=0, keepdims=True)
    k = (cnt + 1) // 2

    def bit_step(j, t):
        cand = t + lax.shift_left(jnp.int32(1), 30 - j)
        pred = jnp.where((d < cand) & oh, 1, 0)
        c1 = jnp.sum(pred, axis=0, keepdims=True)
        return jnp.where(c1 >= k, t, cand)

    T = lax.fori_loop(0, 31, bit_step, jnp.zeros((1, G), jnp.int32))

    strict_m = (d < T) & oh
    n_lt = jnp.sum(jnp.where(strict_m, 1, 0), axis=0, keepdims=True)
    eq_m = (d == T) & oh
    need = k - n_lt
    idx = lax.broadcasted_iota(jnp.int32, (NP, 1), 0)

    def tie_step(j, t):
        cand = t + lax.shift_left(jnp.int32(1), 13 - j)
        pred = jnp.where(eq_m & (idx < cand), 1, 0)
        c2 = jnp.sum(pred, axis=0, keepdims=True)
        return jnp.where(c2 >= need, t, cand)

    T2 = lax.fori_loop(0, 14, tie_step, jnp.zeros((1, G), jnp.int32))
    tie_m = eq_m & (idx <= T2) & (need > 0)
    keep_m = strict_m | tie_m
    keep = jnp.max(jnp.where(keep_m, 1, 0), axis=1, keepdims=True) > 0

    sel_ref[...] = jnp.where(keep, s, 0.0)
    nmo_ref[...] = jnp.where(keep, 1.0, 0.0)


def _select(s, nm, bcol):
    return pl.pallas_call(
        _select_body,
        out_shape=(
            jax.ShapeDtypeStruct((NP, 1), jnp.float32),
            jax.ShapeDtypeStruct((NP, 1), jnp.float32),
        ),
        compiler_params=pltpu.CompilerParams(
            vmem_limit_bytes=100 * 1024 * 1024),
    )(s, nm, bcol)


def _readout_body(y_ref, sel_ref, nmo_ref, b_ref, ohT_ref,
                  h2_ref, read_ref):
    y = y_ref[...]
    sel = sel_ref[...]
    keep_f = nmo_ref[...]
    bcol = b_ref[...]
    ohT = ohT_ref[...]
    h2 = y * sel
    h2_ref[...] = h2
    keep = keep_f > 0.0

    sm = jnp.dot(ohT, h2, preferred_element_type=jnp.float32,
                 precision=jax.lax.Precision.HIGHEST)
    kcol = jnp.dot(ohT, keep_f, preferred_element_type=jnp.float32,
                   precision=jax.lax.Precision.HIGHEST)
    mean = sm / jnp.maximum(kcol, 1.0)

    g64 = lax.broadcasted_iota(jnp.int32, (G, 1), 0)

    def max_step(g, mx):
        m = keep & (bcol == g)
        hm = jnp.where(m, h2, -1e30)
        row = jnp.max(hm, axis=0, keepdims=True)
        return jnp.where(g64 == g, row, mx)

    mx = lax.fori_loop(0, G, max_step, jnp.full((G, D), -1e30, jnp.float32))
    mx = jnp.where(kcol > 0.0, mx, 0.0)
    read_ref[:, 0:D] = mx
    read_ref[:, D:2 * D] = mean


def _readout(y, sel, nmo, bcol, ohT):
    return pl.pallas_call(
        _readout_body,
        out_shape=(
            jax.ShapeDtypeStruct((NP, D), jnp.float32),
            jax.ShapeDtypeStruct((G, 2 * D), jnp.float32),
        ),
        compiler_params=pltpu.CompilerParams(
            vmem_limit_bytes=100 * 1024 * 1024),
    )(y, sel, nmo, bcol, ohT)


def _head_body(r1, r2, r3, r4, r5, w1, b1, w2, b2, w3, b3, o_ref):
    bf = jnp.bfloat16
    z = r1[...] + r2[...] + r3[...] + r4[...] + r5[...]
    z = jnp.maximum(jnp.dot(z.astype(bf), w1[...],
                            preferred_element_type=jnp.float32)
                    + b1[...], 0.0)
    z = jnp.maximum(jnp.dot(z.astype(bf), w2[...],
                            preferred_element_type=jnp.float32)
                    + b2[...], 0.0)
    z = jnp.dot(z.astype(bf), w3[...],
                preferred_element_type=jnp.float32) + b3[...]
    col = lax.broadcasted_iota(jnp.int32, (G, D), 1)
    zm = jnp.where(col < DIM_O, z, -1e30)
    m = jnp.max(zm, axis=1, keepdims=True)
    ex = jnp.where(col < DIM_O, jnp.exp(zm - m), 0.0)
    lse = jnp.log(jnp.sum(ex, axis=1, keepdims=True))
    o_ref[...] = zm - m - lse


def _head(reads, w1T, b1, w2T, b2, w3T, b3):
    return pl.pallas_call(
        _head_body,
        out_shape=jax.ShapeDtypeStruct((G, D), jnp.float32),
    )(*reads, w1T, b1, w2T, b2, w3T, b3)


def kernel(x, params, edge_index, batch):
    f32 = jnp.float32
    xp = jnp.pad(x, ((0, NP - N), (0, 0)))
    bcol = jnp.pad(batch, (0, NP - N)).astype(jnp.int32).reshape(NP, 1)
    nm = jnp.pad(jnp.ones((N, 1), f32), ((0, NP - N), (0, 0)))
    ohT = (bcol.reshape(1, NP) == jnp.arange(G, dtype=jnp.int32)
           .reshape(G, 1)).astype(f32)
    srcp = jnp.pad(edge_index[0], (0, EP - E), constant_values=N)
    dstp = jnp.pad(edge_index[1], (0, EP - E), constant_values=N)
    zeros = jnp.zeros((NP, D), f32)

    bf = jnp.bfloat16
    h = xp
    reads = []
    for i in range(5):
        p = params['conv'][i]
        bn = params['bn'][i]
        w = params['pool_w'][i]
        wn = w.astype(bf).reshape(D, 1)
        norm = jnp.linalg.norm(w).reshape(1, 1)

        parts = _sc_agg(h, srcp, dstp, zeros)
        y, s = _conv(parts[0], parts[1], h,
                     p['Wr'].T.astype(bf), p['Wo'].T.astype(bf),
                     p['br'].reshape(1, D), bn['m'].reshape(1, D),
                     bn['v'].reshape(1, D), bn['w'].reshape(1, D),
                     bn['b'].reshape(1, D), wn, norm)
        sel, nmo = _select(s, nm, bcol)
        h, rd = _readout(y, sel, nmo, bcol, ohT)
        nm = nmo
        reads.append(rd)

    l1, l2, l3 = params['lin1'], params['lin2'], params['lin3']
    w2T = jnp.zeros((D, D), f32).at[:, :64].set(l2['W'].T)
    b2 = jnp.zeros((1, D), f32).at[0, :64].set(l2['b'])
    w3T = jnp.zeros((D, D), f32).at[:64, :DIM_O].set(l3['W'].T)
    b3 = jnp.zeros((1, D), f32).at[0, :DIM_O].set(l3['b'])
    out = _head(reads, l1['W'].T.astype(bf), l1['b'].reshape(1, D),
                w2T.astype(bf), b2, w3T.astype(bf), b3)
    return out[:, :DIM_O]
